# baseline (speedup 1.0000x reference)
import functools

import jax
import jax.numpy as jnp
from jax import lax
from jax.experimental import pallas as pl
from jax.experimental.pallas import tpu as pltpu
from jax.experimental.pallas import tpu_sc as plsc

EMBED = 128
NUM_WORKERS = 32
TOKENS = 4096 * 200
X_ROWS = TOKENS // 128
ROWS_PER_WORKER = X_ROWS // NUM_WORKERS
NBUF = 4
NGROUP = (ROWS_PER_WORKER - 4) // NBUF


def _proj_body(e0, e1, e2, e3, w0, w1, w2, w3, o):
    o[0:100] = jnp.dot(e0[...], w0[...], preferred_element_type=jnp.float32)
    o[100:300] = jnp.dot(e1[...], w1[...], preferred_element_type=jnp.float32)
    o[300:600] = jnp.dot(e2[...], w2[...], preferred_element_type=jnp.float32)
    o[600:1000] = jnp.dot(e3[...], w3[...], preferred_element_type=jnp.float32)


def _project_tables(embs, ws):
    return pl.pallas_call(
        _proj_body,
        out_shape=jax.ShapeDtypeStruct((1000, EMBED), jnp.float32),
    )(*embs, *ws)


def _sc_body(x_hbm, tab_hbm, out_hbm, tab_sh, idx_v, rows_v,
             gsem0, gsem1, gsem2, gsem3, ssem0, ssem1, ssem2, ssem3):
    sid = lax.axis_index("s")
    wid = sid * 2 + lax.axis_index("c")
    row0 = wid * ROWS_PER_WORKER
    gsems = (gsem0, gsem1, gsem2, gsem3)
    ssems = (ssem0, ssem1, ssem2, ssem3)

    @pl.when(sid == 0)
    def _():
        pltpu.sync_copy(tab_hbm, tab_sh)

    pltpu.sync_copy(x_hbm.at[pl.ds(row0, ROWS_PER_WORKER)], idx_v)
    plsc.subcore_barrier()

    srcs = (tab_sh, tab_sh, tab_sh, tab_hbm)

    def gissue(j, b):
        pltpu.async_copy(srcs[b].at[idx_v.at[j]], rows_v.at[b], gsems[b])

    def gwait(j, b):
        pltpu.make_async_copy(srcs[b].at[idx_v.at[j]],
                              rows_v.at[b], gsems[b]).wait()

    def sissue(j, b):
        pltpu.async_copy(rows_v.at[b], out_hbm.at[row0 + j], ssems[b])

    def swait(b):
        pltpu.make_async_copy(rows_v.at[b], out_hbm.at[row0],
                              ssems[b]).wait()

    gissue(0, 0)
    gissue(1, 1)
    gwait(0, 0)
    sissue(0, 0)
    gissue(2, 2)
    gwait(1, 1)
    sissue(1, 1)
    gissue(3, 3)

    def group(g, carry):
        j0 = 2 + NBUF * g
        for u in range(NBUF):
            b = (2 + u) % NBUF
            bb = (b + 2) % NBUF
            gwait(j0 + u, b)
            sissue(j0 + u, b)
            swait(bb)
            gissue(j0 + u + 2, bb)
        return carry

    lax.fori_loop(0, NGROUP, group, 0)

    gwait(ROWS_PER_WORKER - 2, 2)
    sissue(ROWS_PER_WORKER - 2, 2)
    gwait(ROWS_PER_WORKER - 1, 3)
    sissue(ROWS_PER_WORKER - 1, 3)
    for b in range(NBUF):
        swait(b)


def kernel(x, emb0, emb1, emb2, emb3, W0, W1, W2, W3):
    table = _project_tables([emb0, emb1, emb2, emb3], [W0, W1, W2, W3])
    x2d = x.reshape(X_ROWS, 128)

    mesh = plsc.VectorSubcoreMesh(core_axis_name="c", subcore_axis_name="s")
    gather = functools.partial(
        pl.kernel,
        mesh=mesh,
        out_type=jax.ShapeDtypeStruct((X_ROWS, 128, EMBED), jnp.float32),
        scratch_types=[
            pltpu.VMEM_SHARED((1000, EMBED), jnp.float32),
            pltpu.VMEM((ROWS_PER_WORKER, 128), jnp.int32),
            pltpu.VMEM((NBUF, 128, EMBED), jnp.float32),
            pltpu.SemaphoreType.DMA,
            pltpu.SemaphoreType.DMA,
            pltpu.SemaphoreType.DMA,
            pltpu.SemaphoreType.DMA,
            pltpu.SemaphoreType.DMA,
            pltpu.SemaphoreType.DMA,
            pltpu.SemaphoreType.DMA,
            pltpu.SemaphoreType.DMA,
        ],
    )(_sc_body)
    out = gather(x2d, table)
    return out.reshape(x.shape + (EMBED,))


# speedup vs baseline: 1.0254x; 1.0254x over previous
import functools

import jax
import jax.numpy as jnp
from jax import lax
from jax.experimental import pallas as pl
from jax.experimental.pallas import tpu as pltpu
from jax.experimental.pallas import tpu_sc as plsc

EMBED = 128
NUM_WORKERS = 32
TOKENS = 4096 * 200
X_ROWS = TOKENS // 128
ROWS_PER_WORKER = X_ROWS // NUM_WORKERS
NBUF = 4
NGROUP = (ROWS_PER_WORKER - 4) // NBUF


NREP = NUM_WORKERS


def _proj_body(e0, e1, e2, e3, w0, w1, w2, w3, o):
    t0 = jnp.dot(e0[...], w0[...], preferred_element_type=jnp.float32)
    t1 = jnp.dot(e1[...], w1[...], preferred_element_type=jnp.float32)
    t2 = jnp.dot(e2[...], w2[...], preferred_element_type=jnp.float32)
    t3 = jnp.dot(e3[...], w3[...], preferred_element_type=jnp.float32)
    for r in range(NREP):
        o[r * 1000 + 0:r * 1000 + 100] = t0
        o[r * 1000 + 100:r * 1000 + 300] = t1
        o[r * 1000 + 300:r * 1000 + 600] = t2
        o[r * 1000 + 600:r * 1000 + 1000] = t3


def _project_tables(embs, ws):
    return pl.pallas_call(
        _proj_body,
        out_shape=jax.ShapeDtypeStruct((NREP * 1000, EMBED), jnp.float32),
    )(*embs, *ws)


def _sc_body(x_hbm, tab_hbm, out_hbm, tab_sh, idx_v, idxh_v, rows_v,
             gsem0, gsem1, gsem2, gsem3, ssem0, ssem1, ssem2, ssem3):
    sid = lax.axis_index("s")
    wid = sid * 2 + lax.axis_index("c")
    row0 = wid * ROWS_PER_WORKER
    gsems = (gsem0, gsem1, gsem2, gsem3)
    ssems = (ssem0, ssem1, ssem2, ssem3)

    @pl.when(sid == 0)
    def _():
        pltpu.sync_copy(tab_hbm.at[pl.ds(0, 1000)], tab_sh)

    pltpu.sync_copy(x_hbm.at[pl.ds(row0, ROWS_PER_WORKER)], idx_v)

    def mkoff(m, carry):
        for k in range(8):
            idxh_v[m, pl.ds(16 * k, 16)] = (
                idx_v[3 + 4 * m, pl.ds(16 * k, 16)] + wid * 1000)
        return carry

    lax.fori_loop(0, ROWS_PER_WORKER // 4, mkoff, 0)
    plsc.subcore_barrier()

    def gissue(j, b, m=0):
        if b == 3:
            pltpu.async_copy(tab_hbm.at[idxh_v.at[m]], rows_v.at[b],
                             gsems[b])
        else:
            pltpu.async_copy(tab_sh.at[idx_v.at[j]], rows_v.at[b], gsems[b])

    def gwait(j, b, m=0):
        if b == 3:
            pltpu.make_async_copy(tab_hbm.at[idxh_v.at[m]], rows_v.at[b],
                                  gsems[b]).wait()
        else:
            pltpu.make_async_copy(tab_sh.at[idx_v.at[j]],
                                  rows_v.at[b], gsems[b]).wait()

    def sissue(j, b):
        pltpu.async_copy(rows_v.at[b], out_hbm.at[row0 + j], ssems[b])

    def swait(b):
        pltpu.make_async_copy(rows_v.at[b], out_hbm.at[row0],
                              ssems[b]).wait()

    gissue(0, 0)
    gissue(1, 1)
    gwait(0, 0)
    sissue(0, 0)
    gissue(2, 2)
    gwait(1, 1)
    sissue(1, 1)
    gissue(3, 3)

    def group(g, carry):
        j0 = 2 + NBUF * g
        for u in range(NBUF):
            b = (2 + u) % NBUF
            bb = (b + 2) % NBUF
            gwait(j0 + u, b, g)
            sissue(j0 + u, b)
            swait(bb)
            gissue(j0 + u + 2, bb, g + 1)
        return carry

    lax.fori_loop(0, NGROUP, group, 0)

    gwait(ROWS_PER_WORKER - 2, 2)
    sissue(ROWS_PER_WORKER - 2, 2)
    gwait(ROWS_PER_WORKER - 1, 3, ROWS_PER_WORKER // 4 - 1)
    sissue(ROWS_PER_WORKER - 1, 3)
    for b in range(NBUF):
        swait(b)


def kernel(x, emb0, emb1, emb2, emb3, W0, W1, W2, W3):
    table = _project_tables([emb0, emb1, emb2, emb3], [W0, W1, W2, W3])
    x2d = x.reshape(X_ROWS, 128)

    mesh = plsc.VectorSubcoreMesh(core_axis_name="c", subcore_axis_name="s")
    gather = functools.partial(
        pl.kernel,
        mesh=mesh,
        out_type=jax.ShapeDtypeStruct((X_ROWS, 128, EMBED), jnp.float32),
        scratch_types=[
            pltpu.VMEM_SHARED((1000, EMBED), jnp.float32),
            pltpu.VMEM((ROWS_PER_WORKER, 128), jnp.int32),
            pltpu.VMEM((ROWS_PER_WORKER // 4, 128), jnp.int32),
            pltpu.VMEM((NBUF, 128, EMBED), jnp.float32),
            pltpu.SemaphoreType.DMA,
            pltpu.SemaphoreType.DMA,
            pltpu.SemaphoreType.DMA,
            pltpu.SemaphoreType.DMA,
            pltpu.SemaphoreType.DMA,
            pltpu.SemaphoreType.DMA,
            pltpu.SemaphoreType.DMA,
            pltpu.SemaphoreType.DMA,
        ],
    )(_sc_body)
    out = gather(x2d, table)
    return out.reshape(x.shape + (EMBED,))


# speedup vs baseline: 1.3024x; 1.2702x over previous
import functools

import jax
import jax.numpy as jnp
from jax import lax
from jax.experimental import pallas as pl
from jax.experimental.pallas import tpu as pltpu
from jax.experimental.pallas import tpu_sc as plsc

EMBED = 128
NUM_WORKERS = 32
TOKENS = 4096 * 200
X_ROWS = TOKENS // 128
ROWS_PER_WORKER = X_ROWS // NUM_WORKERS
NBUF = 5
G = 3
NGROUP = 38


def _proj_body(e0, e1, e2, e3, w0, w1, w2, w3, o):
    o[0:100] = jnp.dot(e0[...], w0[...], preferred_element_type=jnp.float32)
    o[100:300] = jnp.dot(e1[...], w1[...], preferred_element_type=jnp.float32)
    o[300:600] = jnp.dot(e2[...], w2[...], preferred_element_type=jnp.float32)
    o[600:1000] = jnp.dot(e3[...], w3[...], preferred_element_type=jnp.float32)


def _project_tables(embs, ws):
    return pl.pallas_call(
        _proj_body,
        out_shape=jax.ShapeDtypeStruct((1000, EMBED), jnp.float32),
    )(*embs, *ws)


def _sc_body(x_hbm, tab_hbm, out_hbm, tab_sh, idx_v, rows_v,
             gsem0, gsem1, gsem2, gsem3, gsem4,
             ssem0, ssem1, ssem2, ssem3, ssem4):
    sid = lax.axis_index("s")
    wid = sid * 2 + lax.axis_index("c")
    row0 = wid * ROWS_PER_WORKER
    gsems = (gsem0, gsem1, gsem2, gsem3, gsem4)
    ssems = (ssem0, ssem1, ssem2, ssem3, ssem4)

    @pl.when(sid == 0)
    def _():
        pltpu.sync_copy(tab_hbm, tab_sh)

    pltpu.sync_copy(x_hbm.at[pl.ds(row0, ROWS_PER_WORKER)], idx_v)
    plsc.subcore_barrier()

    def gissue(j, b):
        pltpu.async_copy(tab_sh.at[idx_v.at[j]], rows_v.at[b], gsems[b])

    def gwait(j, b):
        pltpu.make_async_copy(tab_sh.at[idx_v.at[j]],
                              rows_v.at[b], gsems[b]).wait()

    def sissue(j, b):
        pltpu.async_copy(rows_v.at[b], out_hbm.at[row0 + j], ssems[b])

    def swait(b):
        pltpu.make_async_copy(rows_v.at[b], out_hbm.at[row0],
                              ssems[b]).wait()

    def step(j, b):
        gwait(j, b)
        sissue(j, b)
        bb = (b + G) % NBUF
        swait(bb)
        gissue(j + G, bb)

    for j in range(G):
        gissue(j, j)
    for j in range(G):
        gwait(j, j)
        sissue(j, j)
        if j + G >= NBUF:
            swait((j + G) % NBUF)
        gissue(j + G, (j + G) % NBUF)

    def group(g, carry):
        j0 = G + NBUF * g
        for u in range(NBUF):
            step(j0 + u, (G + u) % NBUF)
        return carry

    lax.fori_loop(0, NGROUP, group, 0)
    for j in range(193, 197):
        step(j, j % NBUF)

    for j in range(ROWS_PER_WORKER - G, ROWS_PER_WORKER):
        gwait(j, j % NBUF)
        sissue(j, j % NBUF)
    for b in range(NBUF):
        swait(b)


def kernel(x, emb0, emb1, emb2, emb3, W0, W1, W2, W3):
    table = _project_tables([emb0, emb1, emb2, emb3], [W0, W1, W2, W3])
    x2d = x.reshape(X_ROWS, 128)

    mesh = plsc.VectorSubcoreMesh(core_axis_name="c", subcore_axis_name="s")
    gather = functools.partial(
        pl.kernel,
        mesh=mesh,
        out_type=jax.ShapeDtypeStruct((X_ROWS, 128, EMBED), jnp.float32),
        scratch_types=[
            pltpu.VMEM_SHARED((1000, EMBED), jnp.float32),
            pltpu.VMEM((ROWS_PER_WORKER, 128), jnp.int32),
            pltpu.VMEM((NBUF, 128, EMBED), jnp.float32),
        ] + [pltpu.SemaphoreType.DMA] * (2 * NBUF),
    )(_sc_body)
    out = gather(x2d, table)
    return out.reshape(x.shape + (EMBED,))
